# Initial kernel scaffold; baseline (speedup 1.0000x reference)
import jax
import jax.numpy as jnp
from jax.experimental import pallas as pl


def kernel(x, edge_index, W, b):
    raise NotImplementedError("write your pallas kernel here")


# speedup vs baseline: 16.6339x; 16.6339x over previous
import functools

import jax
import jax.numpy as jnp
from jax import lax
from jax.experimental import pallas as pl
from jax.experimental.pallas import tpu as pltpu
from jax.experimental.pallas import tpu_sc as plsc

NC = 2
NS = 16
NW = NC * NS
LANES = 16
CHUNK = 128


def _sc_mesh():
    return plsc.VectorSubcoreMesh(core_axis_name="c", subcore_axis_name="s")


def _make_deg_kernel(ch_per_w, n_pad):
    @functools.partial(
        pl.kernel,
        out_type=jax.ShapeDtypeStruct((NW, n_pad), jnp.float32),
        mesh=_sc_mesh(),
        compiler_params=pltpu.CompilerParams(needs_layout_passes=False),
        scratch_types=[
            pltpu.VMEM((ch_per_w, CHUNK), jnp.int32),
            pltpu.VMEM((n_pad,), jnp.float32),
        ],
    )
    def deg_kernel(row_hbm, out_hbm, idx_v, deg_v):
        c = lax.axis_index("c")
        s = lax.axis_index("s")
        wid = s * NC + c
        pltpu.sync_copy(row_hbm.at[wid], idx_v)
        zeros16 = jnp.zeros((LANES,), jnp.float32)

        def zero_body(i, carry):
            deg_v[pl.ds(i * LANES, LANES)] = zeros16
            return carry

        lax.fori_loop(0, n_pad // LANES, zero_body, 0)

        ones16 = jnp.ones((LANES,), jnp.float32)

        def edge_body(j, carry):
            def lane_body(l, carry2):
                idx = idx_v[j, pl.ds(l * LANES, LANES)]
                plsc.addupdate_scatter(deg_v, [idx], ones16)
                return carry2

            return lax.fori_loop(0, CHUNK // LANES, lane_body, carry)

        lax.fori_loop(0, ch_per_w, edge_body, 0)
        pltpu.sync_copy(deg_v, out_hbm.at[wid])

    return deg_kernel


def _make_agg_kernel(ch_per_w, n_pad, d):
    @functools.partial(
        pl.kernel,
        out_type=jax.ShapeDtypeStruct((NC, n_pad, d), jnp.float32),
        mesh=_sc_mesh(),
        compiler_params=pltpu.CompilerParams(needs_layout_passes=False),
        scratch_types=[
            pltpu.VMEM((ch_per_w, CHUNK), jnp.int32),
            pltpu.VMEM((ch_per_w, CHUNK), jnp.int32),
            pltpu.VMEM((CHUNK, d), jnp.float32),
            pltpu.VMEM_SHARED((n_pad, d), jnp.float32),
            pltpu.SemaphoreType.DMA,
        ],
    )
    def agg_kernel(y_hbm, col_hbm, row_hbm, zeros_hbm, out_hbm,
                   col_v, row_v, buf_v, acc_sh, sem):
        c = lax.axis_index("c")
        s = lax.axis_index("s")
        wid = s * NC + c
        rows_per_tile = n_pad // NS
        pltpu.sync_copy(
            zeros_hbm.at[pl.ds(s * rows_per_tile, rows_per_tile)],
            acc_sh.at[pl.ds(s * rows_per_tile, rows_per_tile)],
        )
        pltpu.sync_copy(col_hbm.at[wid], col_v)
        pltpu.sync_copy(row_hbm.at[wid], row_v)
        plsc.subcore_barrier()

        def chunk_body(j, carry):
            pltpu.async_copy(y_hbm.at[col_v.at[j]], buf_v, sem).wait()
            pltpu.sync_copy(buf_v, acc_sh.at[row_v.at[j]], add=True)
            return carry

        lax.fori_loop(0, ch_per_w, chunk_body, 0)
        plsc.subcore_barrier()
        pltpu.sync_copy(
            acc_sh.at[pl.ds(s * rows_per_tile, rows_per_tile)],
            out_hbm.at[c, pl.ds(s * rows_per_tile, rows_per_tile)],
        )

    return agg_kernel


def _prep_body(degp_ref, x_ref, w_ref, y_ref, dis_ref):
    deg = jnp.sum(degp_ref[...], axis=0)
    dis = jnp.where(deg > 0.0, lax.rsqrt(deg), 0.0)
    dis_ref[...] = dis
    n = x_ref.shape[0]
    z = lax.dot_general(
        x_ref[...], w_ref[...],
        (((1,), (1,)), ((), ())),
        preferred_element_type=jnp.float32,
    )
    y_ref[...] = dis[:n, None] * z


def _fin_body(s0_ref, s1_ref, dis_ref, b_ref, o_ref):
    n = o_ref.shape[0]
    ssum = s0_ref[...] + s1_ref[...]
    val = dis_ref[...][:n, None] * ssum + b_ref[...]
    o_ref[...] = jnp.maximum(val, 0.0)


def kernel(x, edge_index, W, b):
    n, d_in = x.shape
    d_out = W.shape[0]
    e = edge_index.shape[1]

    ch_per_w = -(-e // (NW * CHUNK))
    e_pad = NW * ch_per_w * CHUNK
    n_pad = -(-n // (NS * LANES)) * (NS * LANES)

    row = edge_index[0]
    col = edge_index[1]
    pad = e_pad - e
    dump = n + (jnp.arange(pad, dtype=jnp.int32) % (n_pad - n))
    row_p = jnp.concatenate([row, dump]).reshape(NW, ch_per_w, CHUNK)
    col_p = jnp.concatenate(
        [col, jnp.zeros((pad,), jnp.int32)]).reshape(NW, ch_per_w, CHUNK)

    deg_parts = _make_deg_kernel(ch_per_w, n_pad)(row_p)

    y, dis = pl.pallas_call(
        _prep_body,
        out_shape=(
            jax.ShapeDtypeStruct((n, d_in), jnp.float32),
            jax.ShapeDtypeStruct((n_pad,), jnp.float32),
        ),
    )(deg_parts, x, W)

    zeros_nd = jnp.zeros((n_pad, d_in), jnp.float32)
    s_parts = _make_agg_kernel(ch_per_w, n_pad, d_in)(
        y, col_p, row_p, zeros_nd)

    out = pl.pallas_call(
        _fin_body,
        out_shape=jax.ShapeDtypeStruct((n, d_out), jnp.float32),
    )(s_parts[0, :n], s_parts[1, :n], dis, b.reshape(1, d_out))
    return out
